# baseline (speedup 1.0000x reference)
import functools

import jax
import jax.numpy as jnp
from jax import lax
from jax.experimental import pallas as pl
from jax.experimental.pallas import tpu as pltpu
from jax.experimental.pallas import tpu_sc as plsc

N = 10000
E = 320000
DH = 256
NC = 2
NS = 16
NW = NC * NS
NRS = 10112
DUMMY = N
G = 2048
NST = -(-E // G)
EPAD = NST * G
CPT = 8
HPT = 4


def _seg_body(xpta, xptb, srcf, dstf, out, xz, acc, sa, da, sb, db, sem_a, sem_b):
    c = lax.axis_index("c")
    s = lax.axis_index("s")
    w = c * NS + s
    zero16 = jnp.zeros((16,), jnp.float32)

    def za(k, _):
        acc[k // (NRS // 16), pl.ds((k % (NRS // 16)) * 16, 16)] = zero16
        return 0

    lax.fori_loop(0, CPT * (NRS // 16), za, 0)

    for p in range(2):
        tab = xpta if p == 0 else xptb
        pltpu.sync_copy(tab.at[pl.ds(w * CPT, HPT)], xz)

        def process(sidx, didx):
            @plsc.parallel_loop(0, G // 16, unroll=12)
            def _(g):
                s16 = sidx[pl.ds(g * 16, 16)]
                d16 = didx[pl.ds(g * 16, 16)]
                for cc in range(HPT):
                    cc16 = jnp.full((16,), cc, jnp.int32)
                    oc16 = jnp.full((16,), p * HPT + cc, jnp.int32)
                    v = plsc.load_gather(xz, [cc16, s16])
                    plsc.addupdate_scatter(acc, [oc16, d16], v)

        def wait_a():
            pltpu.make_async_copy(srcf.at[0], sa, sem_a).wait()
            pltpu.make_async_copy(dstf.at[0], da, sem_a).wait()

        def wait_b():
            pltpu.make_async_copy(srcf.at[0], sb, sem_b).wait()
            pltpu.make_async_copy(dstf.at[0], db, sem_b).wait()

        pltpu.async_copy(srcf.at[0], sa, sem_a)
        pltpu.async_copy(dstf.at[0], da, sem_a)

        def stage2(i, _):
            t1 = 2 * i + 1
            wait_a()
            pltpu.async_copy(srcf.at[t1], sb, sem_b)
            pltpu.async_copy(dstf.at[t1], db, sem_b)
            process(sa, da)
            wait_b()
            pltpu.async_copy(srcf.at[t1 + 1], sa, sem_a)
            pltpu.async_copy(dstf.at[t1 + 1], da, sem_a)
            process(sb, db)
            return 0

        lax.fori_loop(0, (NST - 1) // 2, stage2, 0)
        wait_a()
        process(sa, da)

    pltpu.sync_copy(acc, out.at[pl.ds(w * CPT, CPT)])


@functools.cache
def _seg_call():
    return pl.kernel(
        _seg_body,
        out_type=jax.ShapeDtypeStruct((DH, NRS), jnp.float32),
        mesh=plsc.VectorSubcoreMesh(
            core_axis_name="c", subcore_axis_name="s", num_cores=NC, num_subcores=NS
        ),
        compiler_params=pltpu.CompilerParams(needs_layout_passes=False),
        scratch_types=[
            pltpu.VMEM((HPT, NRS), jnp.float32),
            pltpu.VMEM((CPT, NRS), jnp.float32),
            pltpu.VMEM((G,), jnp.int32),
            pltpu.VMEM((G,), jnp.int32),
            pltpu.VMEM((G,), jnp.int32),
            pltpu.VMEM((G,), jnp.int32),
            pltpu.SemaphoreType.DMA,
            pltpu.SemaphoreType.DMA,
        ],
    )


def _segment_sum(xpta, xptb, srcf, dstf):
    assert NST % 2 == 1
    return _seg_call()(xpta, xptb, srcf, dstf)


def _cnt_body(dstw, out, cacc, didx):
    c = lax.axis_index("c")
    s = lax.axis_index("s")
    w = c * NS + s
    zero16 = jnp.zeros((16,), jnp.float32)

    def za(k, _):
        cacc[pl.ds(k * 16, 16)] = zero16
        return 0

    lax.fori_loop(0, NRS // 16, za, 0)

    one16 = jnp.ones((16,), jnp.float32)

    def stage(t, _):
        pltpu.sync_copy(dstw.at[w * (NST // NW + 1) + t], didx)

        def grp(g, _):
            d16 = didx[pl.ds(g * 16, 16)]
            plsc.addupdate_scatter(cacc, [d16], one16)
            return 0

        lax.fori_loop(0, G // 16, grp, 0)
        return 0

    lax.fori_loop(0, NST // NW + 1, stage, 0)
    pltpu.sync_copy(cacc, out.at[w])


@functools.cache
def _cnt_call():
    return pl.kernel(
        _cnt_body,
        out_type=jax.ShapeDtypeStruct((NW, NRS), jnp.float32),
        mesh=plsc.VectorSubcoreMesh(
            core_axis_name="c", subcore_axis_name="s", num_cores=NC, num_subcores=NS
        ),
        compiler_params=pltpu.CompilerParams(needs_layout_passes=False),
        scratch_types=[
            pltpu.VMEM((NRS,), jnp.float32),
            pltpu.VMEM((G,), jnp.int32),
        ],
    )


def _count(dstw):
    return _cnt_call()(dstw)


def _emit_mm_outputs(d, xr_ref, a_ref, b_ref):
    xr_ref[...] = d[:, DH:]
    xpt = d[:, :DH].T
    a_ref[...] = xpt
    b_ref[...] = jnp.concatenate(
        [xpt[HPT:], jnp.zeros((HPT, xpt.shape[1]), jnp.float32)], axis=0)


def _mm_body(x_ref, w_ref, xr_ref, a_ref, b_ref):
    d = jnp.dot(x_ref[...], w_ref[...], preferred_element_type=jnp.float32)
    _emit_mm_outputs(d, xr_ref, a_ref, b_ref)


_MM_OUT_SPECS = [
    pl.BlockSpec((128, DH), lambda i: (i, 0)),
    pl.BlockSpec((DH, 128), lambda i: (0, i)),
    pl.BlockSpec((DH, 128), lambda i: (0, i)),
]
_MM_OUT_SHAPE = [
    jax.ShapeDtypeStruct((N, DH), jnp.float32),
    jax.ShapeDtypeStruct((DH, NRS), jnp.float32),
    jax.ShapeDtypeStruct((DH, NRS), jnp.float32),
]


def _mm(x, w):
    k = x.shape[1]
    return pl.pallas_call(
        _mm_body,
        grid=(NRS // 128,),
        in_specs=[
            pl.BlockSpec((128, k), lambda i: (i, 0)),
            pl.BlockSpec((k, 2 * DH), lambda i: (0, 0)),
        ],
        out_specs=_MM_OUT_SPECS,
        out_shape=_MM_OUT_SHAPE,
    )(x, w)


def _post_body(s_ref, cp_ref, xr_ref, b_ref, t_ref, st_ref):
    i = pl.program_id(0)
    cnt = jnp.sum(cp_ref[...], axis=1, keepdims=True)
    inv = 1.0 / jnp.maximum(cnt, 1.0)
    o = s_ref[...] * inv + xr_ref[...] + b_ref[...]
    nrm = jnp.sqrt(jnp.sum(o * o, axis=1, keepdims=True))
    o = o / jnp.maximum(nrm, 1e-12)
    t = jnp.where(o > 0, o, jnp.exp(o) - 1.0)
    t_ref[...] = t

    @pl.when(i == 0)
    def _():
        st_ref[...] = jnp.zeros_like(st_ref)

    st_ref[...] += jnp.concatenate(
        [jnp.sum(t, axis=0, keepdims=True), jnp.sum(t * t, axis=0, keepdims=True)], axis=0
    )


def _post(s, cp, xr, b, bm=400):
    return pl.pallas_call(
        _post_body,
        grid=(N // bm,),
        in_specs=[
            pl.BlockSpec((bm, DH), lambda i: (i, 0)),
            pl.BlockSpec((bm, NW), lambda i: (i, 0)),
            pl.BlockSpec((bm, DH), lambda i: (i, 0)),
            pl.BlockSpec((1, DH), lambda i: (0, 0)),
        ],
        out_specs=[
            pl.BlockSpec((bm, DH), lambda i: (i, 0)),
            pl.BlockSpec((2, DH), lambda i: (0, 0)),
        ],
        out_shape=[
            jax.ShapeDtypeStruct((N, DH), jnp.float32),
            jax.ShapeDtypeStruct((2, DH), jnp.float32),
        ],
    )(s, cp, xr, b)


def _bn_scale_shift(st_ref, g_ref, be_ref):
    mean = st_ref[0:1, :] / N
    var = st_ref[1:2, :] / N - mean * mean
    scale = g_ref[...] * lax.rsqrt(var + 1e-5)
    shift = be_ref[...] - mean * scale
    return scale, shift


def _bn_mm_body(t_ref, st_ref, g_ref, be_ref, w_ref, xr_ref, a_ref, b_ref):
    scale, shift = _bn_scale_shift(st_ref, g_ref, be_ref)
    h = t_ref[...] * scale + shift
    d = jnp.dot(h, w_ref[...], preferred_element_type=jnp.float32)
    _emit_mm_outputs(d, xr_ref, a_ref, b_ref)


def _bn_mm(t, st, g, be, w):
    return pl.pallas_call(
        _bn_mm_body,
        grid=(NRS // 128,),
        in_specs=[
            pl.BlockSpec((128, DH), lambda i: (i, 0)),
            pl.BlockSpec((2, DH), lambda i: (0, 0)),
            pl.BlockSpec((1, DH), lambda i: (0, 0)),
            pl.BlockSpec((1, DH), lambda i: (0, 0)),
            pl.BlockSpec((DH, 2 * DH), lambda i: (0, 0)),
        ],
        out_specs=_MM_OUT_SPECS,
        out_shape=_MM_OUT_SHAPE,
    )(t, st, g, be, w)


def _head_body(t_ref, st_ref, g_ref, be_ref, w1_ref, b1_ref, w2_ref, b2_ref,
               w3_ref, b3_ref, o_ref):
    scale, shift = _bn_scale_shift(st_ref, g_ref, be_ref)
    h = t_ref[...] * scale + shift
    h = jnp.dot(h, w1_ref[...], preferred_element_type=jnp.float32) + b1_ref[...]
    h = jnp.where(h > 0, h, jnp.exp(h) - 1.0)
    h = jnp.dot(h, w2_ref[...], preferred_element_type=jnp.float32) + b2_ref[...]
    h = jnp.where(h > 0, h, jnp.exp(h) - 1.0)
    o_ref[...] = jnp.dot(h, w3_ref[...], preferred_element_type=jnp.float32) + b3_ref[...]


def _head(t, st, g, be, w1, b1, w2, b2, w3, b3, bm, dout):
    return pl.pallas_call(
        _head_body,
        grid=(N // bm,),
        in_specs=[
            pl.BlockSpec((bm, DH), lambda i: (i, 0)),
            pl.BlockSpec((2, DH), lambda i: (0, 0)),
            pl.BlockSpec((1, DH), lambda i: (0, 0)),
            pl.BlockSpec((1, DH), lambda i: (0, 0)),
            pl.BlockSpec((DH, DH), lambda i: (0, 0)),
            pl.BlockSpec((1, DH), lambda i: (0, 0)),
            pl.BlockSpec((DH, DH), lambda i: (0, 0)),
            pl.BlockSpec((1, DH), lambda i: (0, 0)),
            pl.BlockSpec((DH, dout), lambda i: (0, 0)),
            pl.BlockSpec((1, dout), lambda i: (0, 0)),
        ],
        out_specs=pl.BlockSpec((bm, dout), lambda i: (i, 0)),
        out_shape=jax.ShapeDtypeStruct((N, dout), jnp.float32),
    )(t, st, g, be, w1, b1, w2, b2, w3, b3)


def kernel(x, edges, W1l, W1r, b1, W2l, W2r, b2, W3l, W3r, b3,
           g1, be1, g2, be2, g3, be3, Wf1, bf1, Wf2, bf2, Wf3, bf3):
    src = edges[0].astype(jnp.int32)
    dst = edges[1].astype(jnp.int32)
    pad = EPAD - E
    srcf = jnp.concatenate([src, jnp.zeros((pad,), jnp.int32)]).reshape(NST, G)
    dstf = jnp.concatenate([dst, jnp.full((pad,), DUMMY, jnp.int32)]).reshape(NST, G)
    nst_w = NST // NW + 1
    padw = nst_w * NW * G - E
    dstw = jnp.concatenate([dst, jnp.full((padw,), DUMMY, jnp.int32)]).reshape(NW * nst_w, G)

    cpt = _count(dstw).T
    row = lambda v: v.reshape(1, -1)

    def sage_layer(mm_out, b):
        xr, xpta, xptb = mm_out
        st_ = _segment_sum(xpta, xptb, srcf, dstf)
        return _post(st_.T[:N], cpt, xr, b)

    t1, st1 = sage_layer(_mm(x, jnp.concatenate([W1l, W1r], axis=1)), row(b1))
    t2, st2 = sage_layer(
        _bn_mm(t1, st1, row(g1), row(be1), jnp.concatenate([W2l, W2r], axis=1)), row(b2))
    t3, st3 = sage_layer(
        _bn_mm(t2, st2, row(g2), row(be2), jnp.concatenate([W3l, W3r], axis=1)), row(b3))
    return _head(t3, st3, row(g3), row(be3), Wf1, row(bf1), Wf2, row(bf2),
                 Wf3, row(bf3), 400, Wf3.shape[1])


# speedup vs baseline: 1.0958x; 1.0958x over previous
import functools

import jax
import jax.numpy as jnp
from jax import lax
from jax.experimental import pallas as pl
from jax.experimental.pallas import tpu as pltpu
from jax.experimental.pallas import tpu_sc as plsc

N = 10000
E = 320000
DH = 256
NC = 2
NS = 16
NW = NC * NS
NRS = 10112
DUMMY = N
G = 2048
NST = -(-E // G)
EPAD = NST * G
CPT = 8
HPT = 4


def _seg_body(xpta, xptb, srcf, dstf, out, xz, acc, sa, da, sb, db, sem_a, sem_b):
    c = lax.axis_index("c")
    s = lax.axis_index("s")
    w = c * NS + s
    zero16 = jnp.zeros((16,), jnp.float32)

    def za(k, _):
        acc[k // (NRS // 16), pl.ds((k % (NRS // 16)) * 16, 16)] = zero16
        return 0

    lax.fori_loop(0, CPT * (NRS // 16), za, 0)

    for p in range(2):
        tab = xpta if p == 0 else xptb
        pltpu.sync_copy(tab.at[pl.ds(w * CPT, HPT)], xz)

        def process(sidx, didx):
            @plsc.parallel_loop(0, G // 16, unroll=8)
            def _(g):
                s16 = sidx[pl.ds(g * 16, 16)]
                d16 = didx[pl.ds(g * 16, 16)]
                for cc in range(HPT):
                    cc16 = jnp.full((16,), cc, jnp.int32)
                    oc16 = jnp.full((16,), p * HPT + cc, jnp.int32)
                    v = plsc.load_gather(xz, [cc16, s16])
                    plsc.addupdate_scatter(acc, [oc16, d16], v)

        def wait_a():
            pltpu.make_async_copy(srcf.at[0], sa, sem_a).wait()
            pltpu.make_async_copy(dstf.at[0], da, sem_a).wait()

        def wait_b():
            pltpu.make_async_copy(srcf.at[0], sb, sem_b).wait()
            pltpu.make_async_copy(dstf.at[0], db, sem_b).wait()

        pltpu.async_copy(srcf.at[0], sa, sem_a)
        pltpu.async_copy(dstf.at[0], da, sem_a)

        def stage2(i, _):
            t1 = 2 * i + 1
            wait_a()
            pltpu.async_copy(srcf.at[t1], sb, sem_b)
            pltpu.async_copy(dstf.at[t1], db, sem_b)
            process(sa, da)
            wait_b()
            pltpu.async_copy(srcf.at[t1 + 1], sa, sem_a)
            pltpu.async_copy(dstf.at[t1 + 1], da, sem_a)
            process(sb, db)
            return 0

        lax.fori_loop(0, (NST - 1) // 2, stage2, 0)
        wait_a()
        process(sa, da)

    pltpu.sync_copy(acc, out.at[pl.ds(w * CPT, CPT)])


@functools.cache
def _seg_call():
    return pl.kernel(
        _seg_body,
        out_type=jax.ShapeDtypeStruct((DH, NRS), jnp.float32),
        mesh=plsc.VectorSubcoreMesh(
            core_axis_name="c", subcore_axis_name="s", num_cores=NC, num_subcores=NS
        ),
        compiler_params=pltpu.CompilerParams(needs_layout_passes=False),
        scratch_types=[
            pltpu.VMEM((HPT, NRS), jnp.float32),
            pltpu.VMEM((CPT, NRS), jnp.float32),
            pltpu.VMEM((G,), jnp.int32),
            pltpu.VMEM((G,), jnp.int32),
            pltpu.VMEM((G,), jnp.int32),
            pltpu.VMEM((G,), jnp.int32),
            pltpu.SemaphoreType.DMA,
            pltpu.SemaphoreType.DMA,
        ],
    )


def _segment_sum(xpta, xptb, srcf, dstf):
    assert NST % 2 == 1
    return _seg_call()(xpta, xptb, srcf, dstf)


def _cnt_body(dstw, out, cacc, didx):
    c = lax.axis_index("c")
    s = lax.axis_index("s")
    w = c * NS + s
    zero16 = jnp.zeros((16,), jnp.float32)

    def za(k, _):
        cacc[pl.ds(k * 16, 16)] = zero16
        return 0

    lax.fori_loop(0, NRS // 16, za, 0)

    one16 = jnp.ones((16,), jnp.float32)

    def stage(t, _):
        pltpu.sync_copy(dstw.at[w * (NST // NW + 1) + t], didx)

        def grp(g, _):
            d16 = didx[pl.ds(g * 16, 16)]
            plsc.addupdate_scatter(cacc, [d16], one16)
            return 0

        lax.fori_loop(0, G // 16, grp, 0)
        return 0

    lax.fori_loop(0, NST // NW + 1, stage, 0)
    pltpu.sync_copy(cacc, out.at[w])


@functools.cache
def _cnt_call():
    return pl.kernel(
        _cnt_body,
        out_type=jax.ShapeDtypeStruct((NW, NRS), jnp.float32),
        mesh=plsc.VectorSubcoreMesh(
            core_axis_name="c", subcore_axis_name="s", num_cores=NC, num_subcores=NS
        ),
        compiler_params=pltpu.CompilerParams(needs_layout_passes=False),
        scratch_types=[
            pltpu.VMEM((NRS,), jnp.float32),
            pltpu.VMEM((G,), jnp.int32),
        ],
    )


def _count(dstw):
    return _cnt_call()(dstw)


def _emit_mm_outputs(d, xr_ref, a_ref, b_ref):
    xr_ref[...] = d[:, DH:]
    xpt = d[:, :DH].T
    a_ref[...] = xpt
    b_ref[...] = jnp.concatenate(
        [xpt[HPT:], jnp.zeros((HPT, xpt.shape[1]), jnp.float32)], axis=0)


def _mm_body(x_ref, w_ref, xr_ref, a_ref, b_ref):
    d = jnp.dot(x_ref[...], w_ref[...], preferred_element_type=jnp.float32)
    _emit_mm_outputs(d, xr_ref, a_ref, b_ref)


_MM_OUT_SPECS = [
    pl.BlockSpec((128, DH), lambda i: (i, 0)),
    pl.BlockSpec((DH, 128), lambda i: (0, i)),
    pl.BlockSpec((DH, 128), lambda i: (0, i)),
]
_MM_OUT_SHAPE = [
    jax.ShapeDtypeStruct((N, DH), jnp.float32),
    jax.ShapeDtypeStruct((DH, NRS), jnp.float32),
    jax.ShapeDtypeStruct((DH, NRS), jnp.float32),
]


def _mm(x, w):
    k = x.shape[1]
    return pl.pallas_call(
        _mm_body,
        grid=(NRS // 128,),
        in_specs=[
            pl.BlockSpec((128, k), lambda i: (i, 0)),
            pl.BlockSpec((k, 2 * DH), lambda i: (0, 0)),
        ],
        out_specs=_MM_OUT_SPECS,
        out_shape=_MM_OUT_SHAPE,
    )(x, w)


def _post_body(s_ref, cp_ref, xr_ref, b_ref, t_ref, st_ref):
    i = pl.program_id(0)
    cnt = jnp.sum(cp_ref[...], axis=1, keepdims=True)
    inv = 1.0 / jnp.maximum(cnt, 1.0)
    o = s_ref[...] * inv + xr_ref[...] + b_ref[...]
    nrm = jnp.sqrt(jnp.sum(o * o, axis=1, keepdims=True))
    o = o / jnp.maximum(nrm, 1e-12)
    t = jnp.where(o > 0, o, jnp.exp(o) - 1.0)
    t_ref[...] = t

    @pl.when(i == 0)
    def _():
        st_ref[...] = jnp.zeros_like(st_ref)

    st_ref[...] += jnp.concatenate(
        [jnp.sum(t, axis=0, keepdims=True), jnp.sum(t * t, axis=0, keepdims=True)], axis=0
    )


def _post(s, cp, xr, b, bm=400):
    return pl.pallas_call(
        _post_body,
        grid=(N // bm,),
        in_specs=[
            pl.BlockSpec((bm, DH), lambda i: (i, 0)),
            pl.BlockSpec((bm, NW), lambda i: (i, 0)),
            pl.BlockSpec((bm, DH), lambda i: (i, 0)),
            pl.BlockSpec((1, DH), lambda i: (0, 0)),
        ],
        out_specs=[
            pl.BlockSpec((bm, DH), lambda i: (i, 0)),
            pl.BlockSpec((2, DH), lambda i: (0, 0)),
        ],
        out_shape=[
            jax.ShapeDtypeStruct((N, DH), jnp.float32),
            jax.ShapeDtypeStruct((2, DH), jnp.float32),
        ],
    )(s, cp, xr, b)


def _bn_scale_shift(st_ref, g_ref, be_ref):
    mean = st_ref[0:1, :] / N
    var = st_ref[1:2, :] / N - mean * mean
    scale = g_ref[...] * lax.rsqrt(var + 1e-5)
    shift = be_ref[...] - mean * scale
    return scale, shift


def _bn_mm_body(t_ref, st_ref, g_ref, be_ref, w_ref, xr_ref, a_ref, b_ref):
    scale, shift = _bn_scale_shift(st_ref, g_ref, be_ref)
    h = t_ref[...] * scale + shift
    d = jnp.dot(h, w_ref[...], preferred_element_type=jnp.float32)
    _emit_mm_outputs(d, xr_ref, a_ref, b_ref)


def _bn_mm(t, st, g, be, w):
    return pl.pallas_call(
        _bn_mm_body,
        grid=(NRS // 128,),
        in_specs=[
            pl.BlockSpec((128, DH), lambda i: (i, 0)),
            pl.BlockSpec((2, DH), lambda i: (0, 0)),
            pl.BlockSpec((1, DH), lambda i: (0, 0)),
            pl.BlockSpec((1, DH), lambda i: (0, 0)),
            pl.BlockSpec((DH, 2 * DH), lambda i: (0, 0)),
        ],
        out_specs=_MM_OUT_SPECS,
        out_shape=_MM_OUT_SHAPE,
    )(t, st, g, be, w)


def _head_body(t_ref, st_ref, g_ref, be_ref, w1_ref, b1_ref, w2_ref, b2_ref,
               w3_ref, b3_ref, o_ref):
    scale, shift = _bn_scale_shift(st_ref, g_ref, be_ref)
    h = t_ref[...] * scale + shift
    h = jnp.dot(h, w1_ref[...], preferred_element_type=jnp.float32) + b1_ref[...]
    h = jnp.where(h > 0, h, jnp.exp(h) - 1.0)
    h = jnp.dot(h, w2_ref[...], preferred_element_type=jnp.float32) + b2_ref[...]
    h = jnp.where(h > 0, h, jnp.exp(h) - 1.0)
    o_ref[...] = jnp.dot(h, w3_ref[...], preferred_element_type=jnp.float32) + b3_ref[...]


def _head(t, st, g, be, w1, b1, w2, b2, w3, b3, bm, dout):
    return pl.pallas_call(
        _head_body,
        grid=(N // bm,),
        in_specs=[
            pl.BlockSpec((bm, DH), lambda i: (i, 0)),
            pl.BlockSpec((2, DH), lambda i: (0, 0)),
            pl.BlockSpec((1, DH), lambda i: (0, 0)),
            pl.BlockSpec((1, DH), lambda i: (0, 0)),
            pl.BlockSpec((DH, DH), lambda i: (0, 0)),
            pl.BlockSpec((1, DH), lambda i: (0, 0)),
            pl.BlockSpec((DH, DH), lambda i: (0, 0)),
            pl.BlockSpec((1, DH), lambda i: (0, 0)),
            pl.BlockSpec((DH, dout), lambda i: (0, 0)),
            pl.BlockSpec((1, dout), lambda i: (0, 0)),
        ],
        out_specs=pl.BlockSpec((bm, dout), lambda i: (i, 0)),
        out_shape=jax.ShapeDtypeStruct((N, dout), jnp.float32),
    )(t, st, g, be, w1, b1, w2, b2, w3, b3)


def kernel(x, edges, W1l, W1r, b1, W2l, W2r, b2, W3l, W3r, b3,
           g1, be1, g2, be2, g3, be3, Wf1, bf1, Wf2, bf2, Wf3, bf3):
    src = edges[0].astype(jnp.int32)
    dst = edges[1].astype(jnp.int32)
    pad = EPAD - E
    srcf = jnp.concatenate([src, jnp.zeros((pad,), jnp.int32)]).reshape(NST, G)
    dstf = jnp.concatenate([dst, jnp.full((pad,), DUMMY, jnp.int32)]).reshape(NST, G)
    nst_w = NST // NW + 1
    padw = nst_w * NW * G - E
    dstw = jnp.concatenate([dst, jnp.full((padw,), DUMMY, jnp.int32)]).reshape(NW * nst_w, G)

    cpt = _count(dstw).T
    row = lambda v: v.reshape(1, -1)

    def sage_layer(mm_out, b):
        xr, xpta, xptb = mm_out
        st_ = _segment_sum(xpta, xptb, srcf, dstf)
        return _post(st_.T[:N], cpt, xr, b)

    t1, st1 = sage_layer(_mm(x, jnp.concatenate([W1l, W1r], axis=1)), row(b1))
    t2, st2 = sage_layer(
        _bn_mm(t1, st1, row(g1), row(be1), jnp.concatenate([W2l, W2r], axis=1)), row(b2))
    t3, st3 = sage_layer(
        _bn_mm(t2, st2, row(g2), row(be2), jnp.concatenate([W3l, W3r], axis=1)), row(b3))
    return _head(t3, st3, row(g3), row(be3), Wf1, row(bf1), Wf2, row(bf2),
                 Wf3, row(bf3), 400, Wf3.shape[1])
